# baseline (speedup 1.0000x reference)
import jax
import jax.numpy as jnp
from jax import lax
from jax.experimental import pallas as pl
from jax.experimental.pallas import tpu as pltpu
from jax.experimental.pallas import tpu_sc as plsc

N = 50000
E = 800000
HEADS = 4
HID = 16
IN_DIM = 16
FEAT = 128

NC = 2
NS = 16

R = 5000
K = 240
NB = 210
EP = NS * NB * K
N2 = 3128 * NS
RPT = 3128

_f32 = jnp.float32
_bf16 = jnp.bfloat16
_i32 = jnp.int32


def _prep_body(ids_ref, fragp_ref, etp_ref, w1_ref, b1_ref, wg_ref, att_ref,
               xp_ref, as4_ref, ad4_ref, xp0_ref, xp1_ref):
  ids = ids_ref[...]
  onehot = (ids == lax.broadcasted_iota(_i32, (1, 100), 1)).astype(_f32)
  x_in = jnp.dot(onehot, etp_ref[...], preferred_element_type=_f32)
  x_in = x_in + fragp_ref[...]
  x = jnp.dot(x_in, w1_ref[...], preferred_element_type=_f32) + b1_ref[...]
  xp = jnp.dot(x, wg_ref[...], preferred_element_type=_f32)
  xp_ref[...] = xp
  xp0_ref[...] = xp[:, 0:32].astype(_bf16)
  xp1_ref[...] = xp[:, 32:64].astype(_bf16)
  hh = lax.broadcasted_iota(_i32, (64, 16), 0) // HID
  jj = lax.broadcasted_iota(_i32, (64, 16), 1)
  sel4 = (hh == jj).astype(_f32)
  att = att_ref[...]
  as4_ref[...] = jnp.dot(xp * att[0:1, :], sel4, preferred_element_type=_f32)
  ad4_ref[...] = jnp.dot(xp * att[1:2, :], sel4, preferred_element_type=_f32)


def _prep(ids2d, fragp, etp, w1, b1, wg, att):
  return pl.pallas_call(
      _prep_body,
      grid=(N // R,),
      in_specs=[
          pl.BlockSpec((R, 1), lambda i: (i, 0)),
          pl.BlockSpec((R, IN_DIM), lambda i: (i, 0)),
          pl.BlockSpec((100, IN_DIM), lambda i: (0, 0)),
          pl.BlockSpec((IN_DIM, IN_DIM), lambda i: (0, 0)),
          pl.BlockSpec((1, IN_DIM), lambda i: (0, 0)),
          pl.BlockSpec((IN_DIM, 64), lambda i: (0, 0)),
          pl.BlockSpec((2, 64), lambda i: (0, 0)),
      ],
      out_specs=[
          pl.BlockSpec((R, 64), lambda i: (i, 0)),
          pl.BlockSpec((R, 16), lambda i: (i, 0)),
          pl.BlockSpec((R, 16), lambda i: (i, 0)),
          pl.BlockSpec((R, 32), lambda i: (i, 0)),
          pl.BlockSpec((R, 32), lambda i: (i, 0)),
      ],
      out_shape=[
          jax.ShapeDtypeStruct((N, 64), _f32),
          jax.ShapeDtypeStruct((N, 16), _f32),
          jax.ShapeDtypeStruct((N, 16), _f32),
          jax.ShapeDtypeStruct((N, 32), _bf16),
          jax.ShapeDtypeStruct((N, 32), _bf16),
      ],
  )(ids2d, fragp, etp, w1, b1, wg, att)


def _sc_body(src_hbm, dst_hbm, dstg_hbm, as4_hbm, ad4_hbm, xp0_hbm, xp1_hbm,
             zs_hbm, zm_hbm,
             s_out, m_out,
             sidxA, didxA, didxgA, srowA, drowA, xrA,
             sidxB, didxB, didxgB, srowB, drowB, xrB,
             msgb, accs, accm, semA, semB):
  c = lax.axis_index("c")
  s = lax.axis_index("s")
  r0 = s * RPT
  pltpu.sync_copy(zs_hbm.at[pl.ds(r0, RPT)], accs.at[pl.ds(r0, RPT)])
  pltpu.sync_copy(zm_hbm.at[pl.ds(r0, RPT)], accm.at[pl.ds(r0, RPT)])
  plsc.subcore_barrier()

  base0 = s * (NB * K)

  def issue(j, sidx, didx, didxg, srow, drow, xr, sem):
    base = base0 + j * K
    pltpu.sync_copy(src_hbm.at[pl.ds(base, K)], sidx)
    pltpu.sync_copy(dst_hbm.at[pl.ds(base, K)], didx)
    pltpu.sync_copy(dstg_hbm.at[pl.ds(base, K)], didxg)
    pltpu.async_copy(as4_hbm.at[sidx], srow, sem)
    pltpu.async_copy(ad4_hbm.at[didxg], drow, sem)

    @pl.when(c == 0)
    def _():
      pltpu.async_copy(xp0_hbm.at[sidx], xr, sem)

    @pl.when(c == 1)
    def _():
      pltpu.async_copy(xp1_hbm.at[sidx], xr, sem)

  def drain(sidx, didxg, srow, drow, xr, sem):
    pltpu.make_async_copy(as4_hbm.at[sidx], srow, sem).wait()
    pltpu.make_async_copy(ad4_hbm.at[didxg], drow, sem).wait()
    pltpu.make_async_copy(xp0_hbm.at[sidx], xr, sem).wait()

  idxc = (lax.broadcasted_iota(_i32, (16,), 0) >= 8).astype(_i32) + 2 * c

  def compute(didx, srow, drow, xr):
    def edge8(k8, carry2):
      k0 = k8 * 8
      for u in range(8):
        k = k0 + u
        z = srow[k] + drow[k]
        t = jnp.exp(jnp.where(z >= 0.0, z, 0.2 * z))
        drow[k] = t
        m8 = jnp.take_along_axis(t, idxc, axis=0)
        a, b = plsc.unpack(xr[k], format=plsc.PackFormat.INTERLEAVED)
        msgb[k] = plsc.pack(a * m8, b * m8,
                            format=plsc.PackFormat.INTERLEAVED)
      return carry2

    lax.fori_loop(0, K // 8, edge8, 0)
    pltpu.sync_copy(drow, accs.at[didx], add=True)
    pltpu.sync_copy(msgb, accm.at[didx], add=True)

  issue(0, sidxA, didxA, didxgA, srowA, drowA, xrA, semA)

  def outer(i, carry):
    jj = i * 2
    issue(jj + 1, sidxB, didxB, didxgB, srowB, drowB, xrB, semB)
    drain(sidxA, didxgA, srowA, drowA, xrA, semA)
    compute(didxA, srowA, drowA, xrA)
    issue(jj + 2, sidxA, didxA, didxgA, srowA, drowA, xrA, semA)
    drain(sidxB, didxgB, srowB, drowB, xrB, semB)
    compute(didxB, srowB, drowB, xrB)
    return carry

  lax.fori_loop(0, NB // 2 - 1, outer, 0)
  issue(NB - 1, sidxB, didxB, didxgB, srowB, drowB, xrB, semB)
  drain(sidxA, didxgA, srowA, drowA, xrA, semA)
  compute(didxA, srowA, drowA, xrA)
  drain(sidxB, didxgB, srowB, drowB, xrB, semB)
  compute(didxB, srowB, drowB, xrB)

  plsc.subcore_barrier()
  pltpu.sync_copy(accs.at[pl.ds(r0, RPT)], s_out.at[c, pl.ds(r0, RPT)])
  pltpu.sync_copy(accm.at[pl.ds(r0, RPT)], m_out.at[c, pl.ds(r0, RPT)])


def _sc_edges(src, dst, dstg, as4, ad4, xp0, xp1, zs, zm):
  mesh = plsc.VectorSubcoreMesh(core_axis_name="c", subcore_axis_name="s")
  f = pl.kernel(
      _sc_body,
      out_type=(
          jax.ShapeDtypeStruct((NC, N2, 16), _f32),
          jax.ShapeDtypeStruct((NC, N2, 32), _bf16),
      ),
      mesh=mesh,
      compiler_params=pltpu.CompilerParams(
          use_tc_tiling_on_sc=False, needs_layout_passes=False),
      scratch_types=[
          pltpu.VMEM((K,), _i32),
          pltpu.VMEM((K,), _i32),
          pltpu.VMEM((K,), _i32),
          pltpu.VMEM((K, 16), _f32),
          pltpu.VMEM((K, 16), _f32),
          pltpu.VMEM((K, 32), _bf16),
          pltpu.VMEM((K,), _i32),
          pltpu.VMEM((K,), _i32),
          pltpu.VMEM((K,), _i32),
          pltpu.VMEM((K, 16), _f32),
          pltpu.VMEM((K, 16), _f32),
          pltpu.VMEM((K, 32), _bf16),
          pltpu.VMEM((K, 32), _bf16),
          pltpu.VMEM_SHARED((N2, 16), _f32),
          pltpu.VMEM_SHARED((N2, 32), _bf16),
          pltpu.SemaphoreType.DMA,
          pltpu.SemaphoreType.DMA,
      ],
  )
  return f(src, dst, dstg, as4, ad4, xp0, xp1, zs, zm)


def _post_body(m_ref, s_ref, xp_ref, as4_ref, ad4_ref, bg_ref, w2a_ref,
               w2b_ref, b2_ref, out_ref):
  zp = as4_ref[...] + ad4_ref[...]
  tsp = jnp.exp(jnp.where(zp >= 0.0, zp, 0.2 * zp))
  ts0 = tsp[:, 0:2]
  ts1 = tsp[:, 2:4]
  mm0 = m_ref[0].astype(_f32)
  mm1 = m_ref[1].astype(_f32)
  s0 = s_ref[0][:, 0:2]
  s1 = s_ref[1][:, 2:4]
  hi = lax.broadcasted_iota(_i32, (2, 32), 0)
  ji = lax.broadcasted_iota(_i32, (2, 32), 1) // HID
  exp2 = (hi == ji).astype(_f32)
  xp = xp_ref[...]
  bg = bg_ref[...]
  sr0 = 1.0 / (s0 + ts0 + 1e-16)
  sr1 = 1.0 / (s1 + ts1 + 1e-16)
  den0 = jnp.dot(sr0, exp2, preferred_element_type=_f32)
  den1 = jnp.dot(sr1, exp2, preferred_element_type=_f32)
  tse0 = jnp.dot(ts0, exp2, preferred_element_type=_f32)
  tse1 = jnp.dot(ts1, exp2, preferred_element_type=_f32)
  y0 = (mm0 + tse0 * xp[:, 0:32]) * den0 + bg[0:1, 0:32]
  y1 = (mm1 + tse1 * xp[:, 32:64]) * den1 + bg[0:1, 32:64]
  out = (jnp.dot(y0, w2a_ref[...], preferred_element_type=_f32)
         + jnp.dot(y1, w2b_ref[...], preferred_element_type=_f32)
         + b2_ref[...])
  out_ref[...] = out


def _post(m, s_parts, xp, as4, ad4, bg2d, w2a, w2b, b22d):
  return pl.pallas_call(
      _post_body,
      grid=(N // R,),
      in_specs=[
          pl.BlockSpec((NC, R, 32), lambda i: (0, i, 0)),
          pl.BlockSpec((NC, R, 16), lambda i: (0, i, 0)),
          pl.BlockSpec((R, 64), lambda i: (i, 0)),
          pl.BlockSpec((R, 16), lambda i: (i, 0)),
          pl.BlockSpec((R, 16), lambda i: (i, 0)),
          pl.BlockSpec((1, 64), lambda i: (0, 0)),
          pl.BlockSpec((32, FEAT), lambda i: (0, 0)),
          pl.BlockSpec((32, FEAT), lambda i: (0, 0)),
          pl.BlockSpec((1, FEAT), lambda i: (0, 0)),
      ],
      out_specs=pl.BlockSpec((R, FEAT), lambda i: (i, 0)),
      out_shape=jax.ShapeDtypeStruct((N, FEAT), _f32),
  )(m, s_parts, xp, as4, ad4, bg2d, w2a, w2b, b22d)


def kernel(node_tensor, edge_tensor, frag_bond_tensor, emb_table, W1, b1, Wg,
           att_src, att_dst, bg, W2, b2):
  ids2d = node_tensor.astype(_i32).reshape(N, 1)
  src = jnp.concatenate(
      [edge_tensor[:, 0].astype(_i32), jnp.zeros((EP - E,), _i32)])
  dst_r = edge_tensor[:, 1].astype(_i32)
  dst = jnp.concatenate([dst_r, jnp.full((EP - E,), N, _i32)])
  dstg = jnp.concatenate([dst_r, jnp.zeros((EP - E,), _i32)])
  etp = jnp.pad(emb_table, ((0, 0), (0, IN_DIM - emb_table.shape[1])))
  fragp = jnp.pad(frag_bond_tensor, ((0, 0), (IN_DIM - 3, 0)))
  att = jnp.stack([att_src.reshape(64), att_dst.reshape(64)])

  xp, as4, ad4, xp0, xp1 = _prep(ids2d, fragp, etp, W1, b1.reshape(1, IN_DIM),
                                 Wg, att)

  zs = jnp.zeros((N2, 16), _f32)
  zm = jnp.zeros((N2, 32), _bf16)
  s_parts, m = _sc_edges(src, dst, dstg, as4, ad4, xp0, xp1, zs, zm)

  out = _post(m, s_parts, xp, as4, ad4, bg.reshape(1, 64),
              W2[0:32], W2[32:64], b2.reshape(1, FEAT))
  return out


# speedup vs baseline: 1.0023x; 1.0023x over previous
import jax
import jax.numpy as jnp
from jax import lax
from jax.experimental import pallas as pl
from jax.experimental.pallas import tpu as pltpu
from jax.experimental.pallas import tpu_sc as plsc

N = 50000
E = 800000
HEADS = 4
HID = 16
IN_DIM = 16
FEAT = 128

NC = 2
NS = 16

R = 5000
K = 240
NB = 210
EP = NS * NB * K
N2 = 3128 * NS
RPT = 3128

_f32 = jnp.float32
_bf16 = jnp.bfloat16
_i32 = jnp.int32


def _prep_body(ids_ref, fragp_ref, etp_ref, w1_ref, b1_ref, wg_ref, att_ref,
               xp_ref, as_ref, ad_ref):
  ids = ids_ref[...]
  onehot = (ids == lax.broadcasted_iota(_i32, (1, 100), 1)).astype(_f32)
  x_in = jnp.dot(onehot, etp_ref[...], preferred_element_type=_f32)
  x_in = x_in + fragp_ref[...]
  x = jnp.dot(x_in, w1_ref[...], preferred_element_type=_f32) + b1_ref[...]
  xp = jnp.dot(x, wg_ref[...], preferred_element_type=_f32)
  xp_ref[...] = xp
  hh = lax.broadcasted_iota(_i32, (64, 32), 0) // HID
  jj = lax.broadcasted_iota(_i32, (64, 32), 1)
  sel = (jj == 16 * (hh // 2) + (hh % 2)).astype(_f32)
  att = att_ref[...]
  as_ref[...] = jnp.dot(xp * att[0:1, :], sel, preferred_element_type=_f32)
  ad_ref[...] = jnp.dot(xp * att[1:2, :], sel, preferred_element_type=_f32)


def _prep(ids2d, fragp, etp, w1, b1, wg, att):
  return pl.pallas_call(
      _prep_body,
      grid=(N // R,),
      in_specs=[
          pl.BlockSpec((R, 1), lambda i: (i, 0)),
          pl.BlockSpec((R, IN_DIM), lambda i: (i, 0)),
          pl.BlockSpec((100, IN_DIM), lambda i: (0, 0)),
          pl.BlockSpec((IN_DIM, IN_DIM), lambda i: (0, 0)),
          pl.BlockSpec((1, IN_DIM), lambda i: (0, 0)),
          pl.BlockSpec((IN_DIM, 64), lambda i: (0, 0)),
          pl.BlockSpec((2, 64), lambda i: (0, 0)),
      ],
      out_specs=[
          pl.BlockSpec((R, 64), lambda i: (i, 0)),
          pl.BlockSpec((R, 32), lambda i: (i, 0)),
          pl.BlockSpec((R, 32), lambda i: (i, 0)),
      ],
      out_shape=[
          jax.ShapeDtypeStruct((N, 64), _f32),
          jax.ShapeDtypeStruct((N, 32), _f32),
          jax.ShapeDtypeStruct((N, 32), _f32),
      ],
  )(ids2d, fragp, etp, w1, b1, wg, att)


def _sc_body(src_hbm, dst_hbm, as2_hbm, ad2_hbm, xpb_hbm, zs_hbm, zm_hbm,
             s_out, m_out,
             sidxA, didxA, didx2A, srowA, drowA, xrA,
             sidxB, didxB, didx2B, srowB, drowB, xrB,
             msgb, accs, accm, semA, semB):
  c = lax.axis_index("c")
  s = lax.axis_index("s")
  r0 = s * RPT
  pltpu.sync_copy(zs_hbm.at[pl.ds(r0, RPT)], accs.at[pl.ds(r0, RPT)])
  pltpu.sync_copy(zm_hbm.at[pl.ds(r0, RPT)], accm.at[pl.ds(r0, RPT)])
  plsc.subcore_barrier()

  base0 = s * (NB * K)

  def issue(j, sidx, didx, didx2, srow, drow, xr, sem):
    base = base0 + j * K
    pltpu.sync_copy(src_hbm.at[pl.ds(base, K)], sidx)
    pltpu.sync_copy(dst_hbm.at[pl.ds(base, K)], didx)

    def ichunk(i, carry2):
      sl = pl.ds(i * 16, 16)
      sidx[sl] = sidx[sl] * 2 + c
      didx2[sl] = didx[sl] * 2 + c
      return carry2

    lax.fori_loop(0, K // 16, ichunk, 0)
    pltpu.async_copy(as2_hbm.at[sidx], srow, sem)
    pltpu.async_copy(ad2_hbm.at[didx2], drow, sem)
    pltpu.async_copy(xpb_hbm.at[sidx], xr, sem)

  def drain(sidx, didx2, srow, drow, xr, sem):
    pltpu.make_async_copy(as2_hbm.at[sidx], srow, sem).wait()
    pltpu.make_async_copy(ad2_hbm.at[didx2], drow, sem).wait()
    pltpu.make_async_copy(xpb_hbm.at[sidx], xr, sem).wait()

  idx01 = (lax.broadcasted_iota(_i32, (16,), 0) >= 8).astype(_i32)

  def compute(didx, srow, drow, xr):
    def edge4(k4, carry2):
      k0 = k4 * 8
      for u in range(8):
        k = k0 + u
        z = srow[k] + drow[k]
        t = jnp.exp(jnp.where(z >= 0.0, z, 0.2 * z))
        drow[k] = t
        m8 = jnp.take_along_axis(t, idx01, axis=0)
        a, b = plsc.unpack(xr[k], format=plsc.PackFormat.INTERLEAVED)
        msgb[k] = plsc.pack(a * m8, b * m8,
                            format=plsc.PackFormat.INTERLEAVED)
      return carry2

    lax.fori_loop(0, K // 8, edge4, 0)
    pltpu.sync_copy(drow, accs.at[didx], add=True)
    pltpu.sync_copy(msgb, accm.at[didx], add=True)

  issue(0, sidxA, didxA, didx2A, srowA, drowA, xrA, semA)

  def outer(i, carry):
    jj = i * 2
    issue(jj + 1, sidxB, didxB, didx2B, srowB, drowB, xrB, semB)
    drain(sidxA, didx2A, srowA, drowA, xrA, semA)
    compute(didxA, srowA, drowA, xrA)
    issue(jj + 2, sidxA, didxA, didx2A, srowA, drowA, xrA, semA)
    drain(sidxB, didx2B, srowB, drowB, xrB, semB)
    compute(didxB, srowB, drowB, xrB)
    return carry

  lax.fori_loop(0, NB // 2 - 1, outer, 0)
  issue(NB - 1, sidxB, didxB, didx2B, srowB, drowB, xrB, semB)
  drain(sidxA, didx2A, srowA, drowA, xrA, semA)
  compute(didxA, srowA, drowA, xrA)
  drain(sidxB, didx2B, srowB, drowB, xrB, semB)
  compute(didxB, srowB, drowB, xrB)

  plsc.subcore_barrier()
  pltpu.sync_copy(accs.at[pl.ds(r0, RPT)], s_out.at[c, pl.ds(r0, RPT)])
  pltpu.sync_copy(accm.at[pl.ds(r0, RPT)], m_out.at[c, pl.ds(r0, RPT)])


def _sc_edges(src, dst, as2, ad2, xpb, zs, zm):
  mesh = plsc.VectorSubcoreMesh(core_axis_name="c", subcore_axis_name="s")
  f = pl.kernel(
      _sc_body,
      out_type=(
          jax.ShapeDtypeStruct((NC, N2, 16), _f32),
          jax.ShapeDtypeStruct((NC, N2, 32), _bf16),
      ),
      mesh=mesh,
      compiler_params=pltpu.CompilerParams(
          use_tc_tiling_on_sc=False, needs_layout_passes=False),
      scratch_types=[
          pltpu.VMEM((K,), _i32),
          pltpu.VMEM((K,), _i32),
          pltpu.VMEM((K,), _i32),
          pltpu.VMEM((K, 16), _f32),
          pltpu.VMEM((K, 16), _f32),
          pltpu.VMEM((K, 32), _bf16),
          pltpu.VMEM((K,), _i32),
          pltpu.VMEM((K,), _i32),
          pltpu.VMEM((K,), _i32),
          pltpu.VMEM((K, 16), _f32),
          pltpu.VMEM((K, 16), _f32),
          pltpu.VMEM((K, 32), _bf16),
          pltpu.VMEM((K, 32), _bf16),
          pltpu.VMEM_SHARED((N2, 16), _f32),
          pltpu.VMEM_SHARED((N2, 32), _bf16),
          pltpu.SemaphoreType.DMA,
          pltpu.SemaphoreType.DMA,
      ],
  )
  return f(src, dst, as2, ad2, xpb, zs, zm)


def _post_body(m_ref, s_ref, xp_ref, as_ref, ad_ref, bg_ref, w2a_ref,
               w2b_ref, b2_ref, out_ref):
  asp = as_ref[...]
  adp = ad_ref[...]
  zp = asp + adp
  tsp = jnp.exp(jnp.where(zp >= 0.0, zp, 0.2 * zp))
  ts0 = tsp[:, 0:2]
  ts1 = tsp[:, 16:18]
  mm0 = m_ref[0].astype(_f32)
  mm1 = m_ref[1].astype(_f32)
  s0 = s_ref[0][:, 0:2]
  s1 = s_ref[1][:, 0:2]
  hi = lax.broadcasted_iota(_i32, (2, 32), 0)
  ji = lax.broadcasted_iota(_i32, (2, 32), 1) // HID
  exp2 = (hi == ji).astype(_f32)
  xp = xp_ref[...]
  bg = bg_ref[...]
  sr0 = 1.0 / (s0 + ts0 + 1e-16)
  sr1 = 1.0 / (s1 + ts1 + 1e-16)
  den0 = jnp.dot(sr0, exp2, preferred_element_type=_f32)
  den1 = jnp.dot(sr1, exp2, preferred_element_type=_f32)
  tse0 = jnp.dot(ts0, exp2, preferred_element_type=_f32)
  tse1 = jnp.dot(ts1, exp2, preferred_element_type=_f32)
  y0 = (mm0 + tse0 * xp[:, 0:32]) * den0 + bg[0:1, 0:32]
  y1 = (mm1 + tse1 * xp[:, 32:64]) * den1 + bg[0:1, 32:64]
  out = (jnp.dot(y0, w2a_ref[...], preferred_element_type=_f32)
         + jnp.dot(y1, w2b_ref[...], preferred_element_type=_f32)
         + b2_ref[...])
  out_ref[...] = out


def _post(m, s_parts, xp, asp, adp, bg2d, w2a, w2b, b22d):
  return pl.pallas_call(
      _post_body,
      grid=(N // R,),
      in_specs=[
          pl.BlockSpec((NC, R, 32), lambda i: (0, i, 0)),
          pl.BlockSpec((NC, R, 16), lambda i: (0, i, 0)),
          pl.BlockSpec((R, 64), lambda i: (i, 0)),
          pl.BlockSpec((R, 32), lambda i: (i, 0)),
          pl.BlockSpec((R, 32), lambda i: (i, 0)),
          pl.BlockSpec((1, 64), lambda i: (0, 0)),
          pl.BlockSpec((32, FEAT), lambda i: (0, 0)),
          pl.BlockSpec((32, FEAT), lambda i: (0, 0)),
          pl.BlockSpec((1, FEAT), lambda i: (0, 0)),
      ],
      out_specs=pl.BlockSpec((R, FEAT), lambda i: (i, 0)),
      out_shape=jax.ShapeDtypeStruct((N, FEAT), _f32),
  )(m, s_parts, xp, asp, adp, bg2d, w2a, w2b, b22d)


def kernel(node_tensor, edge_tensor, frag_bond_tensor, emb_table, W1, b1, Wg,
           att_src, att_dst, bg, W2, b2):
  ids2d = node_tensor.astype(_i32).reshape(N, 1)
  src = jnp.concatenate(
      [edge_tensor[:, 0].astype(_i32), jnp.zeros((EP - E,), _i32)])
  dst = jnp.concatenate(
      [edge_tensor[:, 1].astype(_i32), jnp.full((EP - E,), N, _i32)])
  etp = jnp.pad(emb_table, ((0, 0), (0, IN_DIM - emb_table.shape[1])))
  fragp = jnp.pad(frag_bond_tensor, ((0, 0), (IN_DIM - 3, 0)))
  att = jnp.stack([att_src.reshape(64), att_dst.reshape(64)])

  xp, asp, adp = _prep(ids2d, fragp, etp, W1, b1.reshape(1, IN_DIM), Wg, att)

  as2 = asp.reshape(2 * N, 16)
  ad2 = jnp.pad(adp.reshape(2 * N, 16), ((0, 2), (0, 0)))
  xpb = xp.astype(_bf16).reshape(2 * N, 32)

  zs = jnp.zeros((N2, 16), _f32)
  zm = jnp.zeros((N2, 32), _bf16)
  s_parts, m = _sc_edges(src, dst, as2, ad2, xpb, zs, zm)

  out = _post(m, s_parts, xp, asp, adp, bg.reshape(1, 64),
              W2[0:32], W2[32:64], b2.reshape(1, FEAT))
  return out
